# baseline (speedup 1.0000x reference)
import functools

import jax
import jax.numpy as jnp
from jax import lax
from jax.experimental import pallas as pl
from jax.experimental.pallas import tpu as pltpu
from jax.experimental.pallas import tpu_sc as plsc

_H = 128
_E = 160000
_NS = 16
_NCORE = 2
_NW = _NS * _NCORE
_EOWN = _E // _NCORE
_ESTRIPE = _EOWN // _NS
_TRASH = _EOWN
_ACCROWS = _EOWN + 16
_CHUNK = 128
_GB = 40
_ZROWS = 500

_FAMS = (("c5", 5, 20000), ("c6", 6, 20000),
         ("p3", 3, 10000), ("p4", 4, 10000),
         ("p5", 5, 10000), ("p6", 6, 10000))
_KS = (3, 4, 5, 6)


def _cdiv(a, b):
    return (a + b - 1) // b


def _spad(n_slots):
    return _cdiv(n_slots, _CHUNK) * _CHUNK


def _sc_gather(edge_rep, idx_pads):
    mesh = plsc.VectorSubcoreMesh(core_axis_name="c", subcore_axis_name="s")
    out_type = [jax.ShapeDtypeStruct((n, _H), jnp.float32) for _, _, n in _FAMS]
    scratch = ([pltpu.VMEM((_GB * k, _H), jnp.float32) for k in _KS]
               + [pltpu.VMEM((_GB * k,), jnp.int32) for k in _KS]
               + [pltpu.VMEM((_GB, _H), jnp.float32),
                  pltpu.SemaphoreType.DMA])

    @functools.partial(pl.kernel, mesh=mesh, out_type=out_type,
                       scratch_types=scratch,
                       compiler_params=pltpu.CompilerParams(
                           use_tc_tiling_on_sc=False))
    def body(edge_hbm, i0, i1, i2, i3, i4, i5,
             g0, g1, g2, g3, g4, g5,
             r3, r4, r5, r6, x3, x4, x5, x6, gsum, sem):
        idx_hbms = (i0, i1, i2, i3, i4, i5)
        g_hbms = (g0, g1, g2, g3, g4, g5)
        rows_k = {3: r3, 4: r4, 5: r5, 6: r6}
        ix_k = {3: x3, 4: x4, 5: x5, 6: x6}
        c = lax.axis_index("c")
        s = lax.axis_index("s")
        w = c * _NS + s

        def run_family_gather(idx_h, g_h, k, n_sub):
            rows = rows_k[k]
            ig = ix_k[k]
            nslots = _GB * k
            nch = n_sub // _GB

            def chunk_body(t, carry):
                ch = t * _NW + w

                @pl.when(ch < nch)
                def _():
                    pltpu.sync_copy(idx_h.at[pl.ds(ch * nslots, nslots)], ig)
                    pltpu.async_copy(edge_hbm.at[ig], rows, sem).wait()

                    def sub_body(r, carry2):
                        b = r * k
                        for cc in range(_H // 16):
                            v = rows[b, pl.ds(cc * 16, 16)]
                            for j in range(1, k):
                                v = v + rows[b + j, pl.ds(cc * 16, 16)]
                            gsum[r, pl.ds(cc * 16, 16)] = v
                        return carry2

                    lax.fori_loop(0, _GB, sub_body, 0)
                    pltpu.sync_copy(gsum, g_h.at[pl.ds(ch * _GB, _GB), :])

                return carry

            lax.fori_loop(0, _cdiv(nch, _NW), chunk_body, 0)

        for fi, (_, k, n_sub) in enumerate(_FAMS):
            run_family_gather(idx_hbms[fi], g_hbms[fi], k, n_sub)

    return body(edge_rep, *idx_pads)


def _sc_counts(idx_pads):
    mesh = plsc.VectorSubcoreMesh(core_axis_name="c", subcore_axis_name="s")
    out_type = jax.ShapeDtypeStruct((_E, 16), jnp.float32)
    scratch = [pltpu.VMEM((_CHUNK,), jnp.int32),
               pltpu.VMEM((_CHUNK,), jnp.int32),
               pltpu.VMEM((_CHUNK, 16), jnp.float32),
               pltpu.VMEM((_ZROWS, 16), jnp.float32),
               pltpu.VMEM_SHARED((_ACCROWS, 16), jnp.float32),
               pltpu.SemaphoreType.DMA]

    @functools.partial(pl.kernel, mesh=mesh, out_type=out_type,
                       scratch_types=scratch,
                       compiler_params=pltpu.CompilerParams(
                           use_tc_tiling_on_sc=False))
    def body(i0, i1, i2, i3, i4, i5, cnt_hbm,
             idxc, locc, onesb, zbuf, acc, sem):
        idx_hbms = (i0, i1, i2, i3, i4, i5)
        c = lax.axis_index("c")
        s = lax.axis_index("s")
        base_e = c * _EOWN
        lanes = lax.iota(jnp.int32, 16)
        zero_row = jnp.zeros((16,), jnp.float32)

        def fill_z(i, carry):
            zbuf[i, :] = zero_row
            return carry

        lax.fori_loop(0, _ZROWS, fill_z, 0)

        def zero_stripe():
            for t in range(_ESTRIPE // _ZROWS):
                pltpu.sync_copy(zbuf,
                                acc.at[pl.ds(s * _ESTRIPE + t * _ZROWS, _ZROWS), :])

        zero_stripe()
        plsc.subcore_barrier()

        def run_family_count(idx_h, fi, nslots_pad):
            one_hot = jnp.where(lanes == fi, 1.0, 0.0).astype(jnp.float32)

            def refill(i, carry):
                onesb[i, :] = one_hot
                return carry

            lax.fori_loop(0, _CHUNK, refill, 0)
            nch = nslots_pad // _CHUNK

            def chunk_body(t, carry):
                ch = t * _NS + s

                @pl.when(ch < nch)
                def _():
                    pltpu.sync_copy(idx_h.at[pl.ds(ch * _CHUNK, _CHUNK)], idxc)
                    for jj in range(_CHUNK // 16):
                        v = idxc[pl.ds(jj * 16, 16)]
                        lv = v - base_e
                        ok = (lv >= 0) & (lv < _EOWN)
                        locc[pl.ds(jj * 16, 16)] = jnp.where(ok, lv, _TRASH)
                    pltpu.sync_copy(onesb, acc.at[locc], add=True)

                return carry

            lax.fori_loop(0, _cdiv(nch, _NS), chunk_body, 0)

        for fi, (_, k, n_sub) in enumerate(_FAMS):
            run_family_count(idx_hbms[fi], fi, _spad(n_sub * k))
        plsc.subcore_barrier()
        pltpu.sync_copy(
            acc.at[pl.ds(s * _ESTRIPE, _ESTRIPE), :],
            cnt_hbm.at[pl.ds(base_e + s * _ESTRIPE, _ESTRIPE), :])

    return body(*idx_pads)


def _sc_scatter(idx_pads, newts):
    mesh = plsc.VectorSubcoreMesh(core_axis_name="c", subcore_axis_name="s")
    out_type = [jax.ShapeDtypeStruct((16, _E, 16), jnp.float32) for _ in _FAMS]
    scratch = [pltpu.VMEM((_CHUNK,), jnp.int32),
               pltpu.VMEM((_CHUNK,), jnp.int32),
               pltpu.VMEM((_CHUNK,), jnp.int32),
               pltpu.VMEM((_CHUNK, 16), jnp.float32),
               pltpu.VMEM((_ZROWS, 16), jnp.float32),
               pltpu.VMEM_SHARED((_ACCROWS, 16), jnp.float32),
               pltpu.SemaphoreType.DMA]

    @functools.partial(pl.kernel, mesh=mesh, out_type=out_type,
                       scratch_types=scratch,
                       compiler_params=pltpu.CompilerParams(
                           use_tc_tiling_on_sc=False))
    def body(i0, i1, i2, i3, i4, i5, t0, t1, t2, t3, t4, t5,
             s0, s1, s2, s3, s4, s5,
             idxb, locb, subb, rowsb, zbuf, acc, sem):
        idx_hbms = (i0, i1, i2, i3, i4, i5)
        newt_hbms = (t0, t1, t2, t3, t4, t5)
        s_hbms = (s0, s1, s2, s3, s4, s5)
        c = lax.axis_index("c")
        s = lax.axis_index("s")
        base_e = c * _EOWN
        lanes = lax.iota(jnp.int32, 16)
        zero_row = jnp.zeros((16,), jnp.float32)

        def fill_z(i, carry):
            zbuf[i, :] = zero_row
            return carry

        lax.fori_loop(0, _ZROWS, fill_z, 0)

        def zero_stripe():
            for t in range(_ESTRIPE // _ZROWS):
                pltpu.sync_copy(zbuf,
                                acc.at[pl.ds(s * _ESTRIPE + t * _ZROWS, _ZROWS), :])

        zero_stripe()
        plsc.subcore_barrier()

        def run_family(idx_h, newt_h, s_h, k, n_sub, nslots_pad):
            nch = nslots_pad // _CHUNK

            def cb_body(cb, carry):
                def chunk_body(t, carry2):
                    ch = t * _NS + s

                    @pl.when(ch < nch)
                    def _():
                        pltpu.sync_copy(idx_h.at[pl.ds(ch * _CHUNK, _CHUNK)], idxb)
                        for jj in range(_CHUNK // 16):
                            v = idxb[pl.ds(jj * 16, 16)]
                            lv = v - base_e
                            ok = (lv >= 0) & (lv < _EOWN)
                            locb[pl.ds(jj * 16, 16)] = jnp.where(ok, lv, _TRASH)
                            pos = ch * _CHUNK + jj * 16 + lanes
                            sub = jnp.minimum(lax.div(pos, k), n_sub - 1)
                            subb[pl.ds(jj * 16, 16)] = sub
                        pltpu.async_copy(newt_h.at[cb].at[subb], rowsb, sem).wait()
                        pltpu.sync_copy(rowsb, acc.at[locb], add=True)

                    return carry2

                lax.fori_loop(0, _cdiv(nch, _NS), chunk_body, 0)
                plsc.subcore_barrier()
                pltpu.sync_copy(
                    acc.at[pl.ds(s * _ESTRIPE, _ESTRIPE), :],
                    s_h.at[cb, pl.ds(base_e + s * _ESTRIPE, _ESTRIPE), :])
                zero_stripe()
                plsc.subcore_barrier()
                return carry

            lax.fori_loop(0, 2 * _H // 16, cb_body, 0)

        for fi, (_, k, n_sub) in enumerate(_FAMS):
            run_family(idx_hbms[fi], newt_hbms[fi], s_hbms[fi], k, n_sub,
                       _spad(n_sub * k))

    return body(*idx_pads, *newts)


def _sc_repack(strips):
    mesh = plsc.VectorSubcoreMesh(core_axis_name="c", subcore_axis_name="s")
    out_type = [jax.ShapeDtypeStruct((_E, 2 * _H), jnp.float32) for _ in _FAMS]
    rp = 200
    scratch = [pltpu.VMEM((rp, 2 * _H), jnp.float32),
               pltpu.SemaphoreType.DMA]

    @functools.partial(pl.kernel, mesh=mesh, out_type=out_type,
                       scratch_types=scratch,
                       compiler_params=pltpu.CompilerParams(
                           use_tc_tiling_on_sc=False))
    def body(t0, t1, t2, t3, t4, t5, o0, o1, o2, o3, o4, o5, stag, sem):
        strip_hbms = (t0, t1, t2, t3, t4, t5)
        out_hbms = (o0, o1, o2, o3, o4, o5)
        c = lax.axis_index("c")
        s = lax.axis_index("s")
        w = c * _NS + s
        rows_per_w = _E // _NW

        def run_family(strip_h, out_h):
            def chunk_body(i, carry):
                r0 = w * rows_per_w + i * rp
                copies = [pltpu.async_copy(strip_h.at[cb, pl.ds(r0, rp), :],
                                           stag.at[:, pl.ds(cb * 16, 16)], sem)
                          for cb in range(16)]
                for cp in copies:
                    cp.wait()
                pltpu.sync_copy(stag, out_h.at[pl.ds(r0, rp), :])
                return carry

            lax.fori_loop(0, rows_per_w // rp, chunk_body, 0)

        for fi in range(len(_FAMS)):
            run_family(strip_hbms[fi], out_hbms[fi])

    return body(*strips)


def _tc_newf(g, rep, wab, k):
    n = g.shape[0]
    bn = 2000
    grid = n // bn

    def body(g_ref, rep_ref, wab_ref, new_ref, newt_ref):
        wmod = wab_ref[0:_H, :] + wab_ref[_H:2 * _H, :] * (1.0 / k)
        y = jnp.dot(g_ref[...], wmod, preferred_element_type=jnp.float32)
        new = jnp.concatenate([y, rep_ref[...]], axis=1)
        new_ref[...] = new
        for j in range(16):
            newt_ref[j, :, :] = new[:, j * 16:(j + 1) * 16]

    return pl.pallas_call(
        body,
        grid=(grid,),
        in_specs=[pl.BlockSpec((bn, _H), lambda i: (i, 0)),
                  pl.BlockSpec((bn, _H), lambda i: (i, 0)),
                  pl.BlockSpec((2 * _H, _H), lambda i: (0, 0))],
        out_specs=[pl.BlockSpec((bn, 2 * _H), lambda i: (i, 0)),
                   pl.BlockSpec((16, bn, 16), lambda i: (0, i, 0))],
        out_shape=[jax.ShapeDtypeStruct((n, 2 * _H), jnp.float32),
                   jax.ShapeDtypeStruct((16, n, 16), jnp.float32)],
    )(g, rep, wab)


def _tc_edge_mm1(edge_rep, s_fams, cnt, we1):
    be = 2000
    grid = _E // be

    def body(er_ref, s5_ref, s6_ref, p3_ref, p4_ref, p5_ref, p6_ref,
             cnt_ref, w_ref, y_ref, sum_ref, sq_ref):
        cnt_blk = cnt_ref[...]

        def recip(fi):
            cc = cnt_blk[:, fi:fi + 1]
            return 1.0 / jnp.maximum(cc, 1.0)

        s5 = s5_ref[...]
        s6 = s6_ref[...]
        p3 = p3_ref[...]
        p4 = p4_ref[...]
        p5 = p5_ref[...]
        p6 = p6_ref[...]
        cyc_s = s5 + s6
        cyc_n = s5 * recip(0) + s6 * recip(1)
        pat_s = p3 + p4 + p5 + p6
        pat_n = (p3 * recip(2) + p4 * recip(3)
                 + p5 * recip(4) + p6 * recip(5))
        w = w_ref[...]
        y = jnp.dot(er_ref[...], w[0:_H, :], preferred_element_type=jnp.float32)
        y += jnp.dot(cyc_s, w[_H:3 * _H, :], preferred_element_type=jnp.float32)
        y += jnp.dot(cyc_n, w[3 * _H:5 * _H, :], preferred_element_type=jnp.float32)
        y += jnp.dot(pat_s, w[5 * _H:7 * _H, :], preferred_element_type=jnp.float32)
        y += jnp.dot(pat_n, w[7 * _H:9 * _H, :], preferred_element_type=jnp.float32)
        y_ref[...] = y
        ps = jnp.sum(y, axis=0, keepdims=True)
        pq = jnp.sum(y * y, axis=0, keepdims=True)

        @pl.when(pl.program_id(0) == 0)
        def _():
            sum_ref[...] = jnp.zeros_like(sum_ref)
            sq_ref[...] = jnp.zeros_like(sq_ref)

        sum_ref[...] += jnp.broadcast_to(ps, sum_ref.shape)
        sq_ref[...] += jnp.broadcast_to(pq, sq_ref.shape)

    m = we1.shape[1]
    return pl.pallas_call(
        body,
        grid=(grid,),
        in_specs=[pl.BlockSpec((be, _H), lambda i: (i, 0))]
        + [pl.BlockSpec((be, 2 * _H), lambda i: (i, 0)) for _ in range(6)]
        + [pl.BlockSpec((be, 16), lambda i: (i, 0)),
           pl.BlockSpec((9 * _H, m), lambda i: (0, 0))],
        out_specs=[pl.BlockSpec((be, m), lambda i: (i, 0)),
                   pl.BlockSpec((8, m), lambda i: (0, 0)),
                   pl.BlockSpec((8, m), lambda i: (0, 0))],
        out_shape=[jax.ShapeDtypeStruct((_E, m), jnp.float32),
                   jax.ShapeDtypeStruct((8, m), jnp.float32),
                   jax.ShapeDtypeStruct((8, m), jnp.float32)],
    )(edge_rep, *s_fams, cnt, we1)


def _tc_mm_stats(x, w, bn):
    n, kdim = x.shape
    m = w.shape[1]
    grid = n // bn

    def body(x_ref, w_ref, y_ref, sum_ref, sq_ref):
        y = jnp.dot(x_ref[...], w_ref[...], preferred_element_type=jnp.float32)
        y_ref[...] = y
        ps = jnp.sum(y, axis=0, keepdims=True)
        pq = jnp.sum(y * y, axis=0, keepdims=True)

        @pl.when(pl.program_id(0) == 0)
        def _():
            sum_ref[...] = jnp.zeros_like(sum_ref)
            sq_ref[...] = jnp.zeros_like(sq_ref)

        sum_ref[...] += jnp.broadcast_to(ps, sum_ref.shape)
        sq_ref[...] += jnp.broadcast_to(pq, sq_ref.shape)

    return pl.pallas_call(
        body,
        grid=(grid,),
        in_specs=[pl.BlockSpec((bn, kdim), lambda i: (i, 0)),
                  pl.BlockSpec((kdim, m), lambda i: (0, 0))],
        out_specs=[pl.BlockSpec((bn, m), lambda i: (i, 0)),
                   pl.BlockSpec((8, m), lambda i: (0, 0)),
                   pl.BlockSpec((8, m), lambda i: (0, 0))],
        out_shape=[jax.ShapeDtypeStruct((n, m), jnp.float32),
                   jax.ShapeDtypeStruct((8, m), jnp.float32),
                   jax.ShapeDtypeStruct((8, m), jnp.float32)],
    )(x, w)


def _tc_bn_relu_mm_stats(y, ssum, ssq, g, b, w2, bn):
    n, kdim = y.shape
    m = w2.shape[1]
    grid = n // bn
    inv_n = 1.0 / n

    def body(y_ref, s_ref, q_ref, g_ref, b_ref, w_ref, o_ref, sum_ref, sq_ref):
        mu = s_ref[0:1, :] * inv_n
        var = q_ref[0:1, :] * inv_n - mu * mu
        h = (y_ref[...] - mu) * lax.rsqrt(var + 1e-5) * g_ref[0:1, :] + b_ref[0:1, :]
        h = jnp.maximum(h, 0.0)
        y2 = jnp.dot(h, w_ref[...], preferred_element_type=jnp.float32)
        o_ref[...] = y2
        ps = jnp.sum(y2, axis=0, keepdims=True)
        pq = jnp.sum(y2 * y2, axis=0, keepdims=True)

        @pl.when(pl.program_id(0) == 0)
        def _():
            sum_ref[...] = jnp.zeros_like(sum_ref)
            sq_ref[...] = jnp.zeros_like(sq_ref)

        sum_ref[...] += jnp.broadcast_to(ps, sum_ref.shape)
        sq_ref[...] += jnp.broadcast_to(pq, sq_ref.shape)

    return pl.pallas_call(
        body,
        grid=(grid,),
        in_specs=[pl.BlockSpec((bn, kdim), lambda i: (i, 0)),
                  pl.BlockSpec((8, kdim), lambda i: (0, 0)),
                  pl.BlockSpec((8, kdim), lambda i: (0, 0)),
                  pl.BlockSpec((1, kdim), lambda i: (0, 0)),
                  pl.BlockSpec((1, kdim), lambda i: (0, 0)),
                  pl.BlockSpec((kdim, m), lambda i: (0, 0))],
        out_specs=[pl.BlockSpec((bn, m), lambda i: (i, 0)),
                   pl.BlockSpec((8, m), lambda i: (0, 0)),
                   pl.BlockSpec((8, m), lambda i: (0, 0))],
        out_shape=[jax.ShapeDtypeStruct((n, m), jnp.float32),
                   jax.ShapeDtypeStruct((8, m), jnp.float32),
                   jax.ShapeDtypeStruct((8, m), jnp.float32)],
    )(y, ssum, ssq, g, b, w2)


def _tc_bn_relu(y, ssum, ssq, g, b, bn):
    n, m = y.shape
    grid = n // bn
    inv_n = 1.0 / n

    def body(y_ref, s_ref, q_ref, g_ref, b_ref, o_ref):
        mu = s_ref[0:1, :] * inv_n
        var = q_ref[0:1, :] * inv_n - mu * mu
        h = (y_ref[...] - mu) * lax.rsqrt(var + 1e-5) * g_ref[0:1, :] + b_ref[0:1, :]
        o_ref[...] = jnp.maximum(h, 0.0)

    return pl.pallas_call(
        body,
        grid=(grid,),
        in_specs=[pl.BlockSpec((bn, m), lambda i: (i, 0)),
                  pl.BlockSpec((8, m), lambda i: (0, 0)),
                  pl.BlockSpec((8, m), lambda i: (0, 0)),
                  pl.BlockSpec((1, m), lambda i: (0, 0)),
                  pl.BlockSpec((1, m), lambda i: (0, 0))],
        out_specs=pl.BlockSpec((bn, m), lambda i: (i, 0)),
        out_shape=jax.ShapeDtypeStruct((n, m), jnp.float32),
    )(y, ssum, ssq, g, b)


def _mlp(x, w1, g1, b1, w2, g2, b2, bn):
    y1, s1, q1 = _tc_mm_stats(x, w1, bn)
    y2, s2, q2 = _tc_bn_relu_mm_stats(y1, s1, q1, g1, b1, w2, bn)
    return _tc_bn_relu(y2, s2, q2, g2, b2, bn)


def kernel(edge_rep, cycle_rep_5, cycle_rep_6, path_rep_3, path_rep_4,
           path_rep_5, path_rep_6, Wab_c5, Wab_c6, Wab_p3, Wab_p4, Wab_p5,
           Wab_p6, We1, ge1, be1, We2, ge2, be2, Wc1, gc1, bc1, Wc2, gc2, bc2,
           Wp1, gp1, bp1, Wp2, gp2, bp2, cyc5_edge_idx, cyc6_edge_idx,
           path3_edge_idx, path4_edge_idx, path5_edge_idx, path6_edge_idx):
    reps = (cycle_rep_5, cycle_rep_6, path_rep_3, path_rep_4, path_rep_5,
            path_rep_6)
    wabs = (Wab_c5, Wab_c6, Wab_p3, Wab_p4, Wab_p5, Wab_p6)
    idxs = (cyc5_edge_idx, cyc6_edge_idx, path3_edge_idx, path4_edge_idx,
            path5_edge_idx, path6_edge_idx)

    idx_pads = []
    for idx, (_, k, n_sub) in zip(idxs, _FAMS):
        nslots = n_sub * k
        pad = _spad(nslots) - nslots
        idx_pads.append(jnp.concatenate(
            [idx, jnp.full((pad,), _E, jnp.int32)]) if pad else idx)

    gs = _sc_gather(edge_rep, idx_pads)
    cnt = _sc_counts(idx_pads)

    news, newts = [], []
    for g, rep, wab, (_, k, _n) in zip(gs, reps, wabs, _FAMS):
        new, newt = _tc_newf(g, rep, wab, k)
        news.append(new)
        newts.append(newt)

    s_strips = _sc_scatter(idx_pads, newts)
    s_fams = _sc_repack(s_strips)

    y1, s1, q1 = _tc_edge_mm1(edge_rep, s_fams, cnt, We1)
    y2, s2, q2 = _tc_bn_relu_mm_stats(y1, s1, q1,
                                      ge1.reshape(1, -1), be1.reshape(1, -1),
                                      We2, 2000)
    edge_out = _tc_bn_relu(y2, s2, q2,
                           ge2.reshape(1, -1), be2.reshape(1, -1), 2000)

    cyc_outs = [_mlp(news[i], Wc1, gc1.reshape(1, -1), bc1.reshape(1, -1),
                     Wc2, gc2.reshape(1, -1), bc2.reshape(1, -1), 2000)
                for i in range(2)]
    pat_outs = [_mlp(news[i], Wp1, gp1.reshape(1, -1), bp1.reshape(1, -1),
                     Wp2, gp2.reshape(1, -1), bp2.reshape(1, -1), 2000)
                for i in range(2, 6)]

    return (edge_out,) + tuple(cyc_outs) + tuple(pat_outs)


# speedup vs baseline: 1.2839x; 1.2839x over previous
import functools

import jax
import jax.numpy as jnp
from jax import lax
from jax.experimental import pallas as pl
from jax.experimental.pallas import tpu as pltpu
from jax.experimental.pallas import tpu_sc as plsc

_H = 128
_E = 160000
_NS = 16
_NCORE = 2
_NW = _NS * _NCORE
_EOWN = _E // _NCORE
_ESTRIPE = _EOWN // _NS
_TRASH = _EOWN
_ACCROWS = _EOWN + 16
_CHUNK = 128
_GB = 40
_ZROWS = 500

_FAMS = (("c5", 5, 20000), ("c6", 6, 20000),
         ("p3", 3, 10000), ("p4", 4, 10000),
         ("p5", 5, 10000), ("p6", 6, 10000))
_KS = (3, 4, 5, 6)


def _cdiv(a, b):
    return (a + b - 1) // b


def _spad(n_slots):
    return _cdiv(n_slots, _CHUNK) * _CHUNK


def _sc_gather(edge_rep, idx_pads):
    mesh = plsc.VectorSubcoreMesh(core_axis_name="c", subcore_axis_name="s")
    out_type = [jax.ShapeDtypeStruct((n, _H), jnp.float32) for _, _, n in _FAMS]
    scratch = ([pltpu.VMEM((_GB * k, _H), jnp.float32) for k in _KS]
               + [pltpu.VMEM((_GB * k,), jnp.int32) for k in _KS]
               + [pltpu.VMEM((_GB, _H), jnp.float32),
                  pltpu.SemaphoreType.DMA])

    @functools.partial(pl.kernel, mesh=mesh, out_type=out_type,
                       scratch_types=scratch,
                       compiler_params=pltpu.CompilerParams(
                           use_tc_tiling_on_sc=False))
    def body(edge_hbm, i0, i1, i2, i3, i4, i5,
             g0, g1, g2, g3, g4, g5,
             r3, r4, r5, r6, x3, x4, x5, x6, gsum, sem):
        idx_hbms = (i0, i1, i2, i3, i4, i5)
        g_hbms = (g0, g1, g2, g3, g4, g5)
        rows_k = {3: r3, 4: r4, 5: r5, 6: r6}
        ix_k = {3: x3, 4: x4, 5: x5, 6: x6}
        c = lax.axis_index("c")
        s = lax.axis_index("s")
        w = c * _NS + s

        def run_family_gather(idx_h, g_h, k, n_sub):
            rows = rows_k[k]
            ig = ix_k[k]
            nslots = _GB * k
            nch = n_sub // _GB

            def chunk_body(t, carry):
                ch = t * _NW + w

                @pl.when(ch < nch)
                def _():
                    pltpu.sync_copy(idx_h.at[pl.ds(ch * nslots, nslots)], ig)
                    pltpu.async_copy(edge_hbm.at[ig], rows, sem).wait()

                    def sub_body(r, carry2):
                        b = r * k
                        for cc in range(_H // 16):
                            v = rows[b, pl.ds(cc * 16, 16)]
                            for j in range(1, k):
                                v = v + rows[b + j, pl.ds(cc * 16, 16)]
                            gsum[r, pl.ds(cc * 16, 16)] = v
                        return carry2

                    lax.fori_loop(0, _GB, sub_body, 0)
                    pltpu.sync_copy(gsum, g_h.at[pl.ds(ch * _GB, _GB), :])

                return carry

            lax.fori_loop(0, _cdiv(nch, _NW), chunk_body, 0)

        for fi, (_, k, n_sub) in enumerate(_FAMS):
            run_family_gather(idx_hbms[fi], g_hbms[fi], k, n_sub)

    return body(edge_rep, *idx_pads)


def _sc_counts(idx_pads):
    mesh = plsc.VectorSubcoreMesh(core_axis_name="c", subcore_axis_name="s")
    out_type = jax.ShapeDtypeStruct((_E, 16), jnp.float32)
    scratch = [pltpu.VMEM((_CHUNK,), jnp.int32),
               pltpu.VMEM((_CHUNK,), jnp.int32),
               pltpu.VMEM((_CHUNK, 16), jnp.float32),
               pltpu.VMEM((_ZROWS, 16), jnp.float32),
               pltpu.VMEM_SHARED((_ACCROWS, 16), jnp.float32),
               pltpu.SemaphoreType.DMA]

    @functools.partial(pl.kernel, mesh=mesh, out_type=out_type,
                       scratch_types=scratch,
                       compiler_params=pltpu.CompilerParams(
                           use_tc_tiling_on_sc=False))
    def body(i0, i1, i2, i3, i4, i5, cnt_hbm,
             idxc, locc, onesb, zbuf, acc, sem):
        idx_hbms = (i0, i1, i2, i3, i4, i5)
        c = lax.axis_index("c")
        s = lax.axis_index("s")
        base_e = c * _EOWN
        lanes = lax.iota(jnp.int32, 16)
        zero_row = jnp.zeros((16,), jnp.float32)

        def fill_z(i, carry):
            zbuf[i, :] = zero_row
            return carry

        lax.fori_loop(0, _ZROWS, fill_z, 0)

        def zero_stripe():
            for t in range(_ESTRIPE // _ZROWS):
                pltpu.sync_copy(zbuf,
                                acc.at[pl.ds(s * _ESTRIPE + t * _ZROWS, _ZROWS), :])

        zero_stripe()
        plsc.subcore_barrier()

        def run_family_count(idx_h, fi, nslots_pad):
            one_hot = jnp.where(lanes == fi, 1.0, 0.0).astype(jnp.float32)

            def refill(i, carry):
                onesb[i, :] = one_hot
                return carry

            lax.fori_loop(0, _CHUNK, refill, 0)
            nch = nslots_pad // _CHUNK

            def chunk_body(t, carry):
                ch = t * _NS + s

                @pl.when(ch < nch)
                def _():
                    pltpu.sync_copy(idx_h.at[pl.ds(ch * _CHUNK, _CHUNK)], idxc)
                    for jj in range(_CHUNK // 16):
                        v = idxc[pl.ds(jj * 16, 16)]
                        lv = v - base_e
                        ok = (lv >= 0) & (lv < _EOWN)
                        locc[pl.ds(jj * 16, 16)] = jnp.where(ok, lv, _TRASH)
                    pltpu.sync_copy(onesb, acc.at[locc], add=True)

                return carry

            lax.fori_loop(0, _cdiv(nch, _NS), chunk_body, 0)

        for fi, (_, k, n_sub) in enumerate(_FAMS):
            run_family_count(idx_hbms[fi], fi, _spad(n_sub * k))
        plsc.subcore_barrier()
        pltpu.sync_copy(
            acc.at[pl.ds(s * _ESTRIPE, _ESTRIPE), :],
            cnt_hbm.at[pl.ds(base_e + s * _ESTRIPE, _ESTRIPE), :])

    return body(*idx_pads)


def _sc_scatter(idx_pads, newts):
    mesh = plsc.VectorSubcoreMesh(core_axis_name="c", subcore_axis_name="s")
    out_type = [jax.ShapeDtypeStruct((_E, 2 * _H), jnp.float32) for _ in _FAMS]
    max_tile_ch = max(_cdiv(_spad(n * k) // _CHUNK, _NS) for _, k, n in _FAMS)
    scratch = [pltpu.VMEM((_CHUNK,), jnp.int32),
               pltpu.VMEM((max_tile_ch, _CHUNK), jnp.int32),
               pltpu.VMEM((max_tile_ch, _CHUNK), jnp.int32),
               pltpu.VMEM((2, _CHUNK, 16), jnp.float32),
               pltpu.VMEM((_ZROWS, 16), jnp.float32),
               pltpu.VMEM_SHARED((_ACCROWS, 16), jnp.float32),
               pltpu.SemaphoreType.DMA,
               pltpu.SemaphoreType.DMA]

    @functools.partial(pl.kernel, mesh=mesh, out_type=out_type,
                       scratch_types=scratch,
                       compiler_params=pltpu.CompilerParams(
                           use_tc_tiling_on_sc=False))
    def body(i0, i1, i2, i3, i4, i5, t0, t1, t2, t3, t4, t5,
             s0, s1, s2, s3, s4, s5,
             idxb, locb, subb, rowsb, zbuf, acc, sem0, sem1):
        idx_hbms = (i0, i1, i2, i3, i4, i5)
        newt_hbms = (t0, t1, t2, t3, t4, t5)
        s_hbms = (s0, s1, s2, s3, s4, s5)
        sems = (sem0, sem1)
        c = lax.axis_index("c")
        s = lax.axis_index("s")
        base_e = c * _EOWN
        lanes = lax.iota(jnp.int32, 16)
        zero_row = jnp.zeros((16,), jnp.float32)

        def fill_z(i, carry):
            zbuf[i, :] = zero_row
            return carry

        lax.fori_loop(0, _ZROWS, fill_z, 0)

        def zero_stripe():
            for t in range(_ESTRIPE // _ZROWS):
                pltpu.sync_copy(zbuf,
                                acc.at[pl.ds(s * _ESTRIPE + t * _ZROWS, _ZROWS), :])

        zero_stripe()
        plsc.subcore_barrier()

        def run_family(idx_h, newt_h, s_h, k, n_sub, nslots_pad):
            nch = nslots_pad // _CHUNK
            n_tile_ch = _cdiv(nch, _NS)

            def decode_body(t, carry):
                ch = t * _NS + s

                @pl.when(ch < nch)
                def _():
                    pltpu.sync_copy(idx_h.at[pl.ds(ch * _CHUNK, _CHUNK)], idxb)
                    for jj in range(_CHUNK // 16):
                        v = idxb[pl.ds(jj * 16, 16)]
                        lv = v - base_e
                        ok = (lv >= 0) & (lv < _EOWN)
                        locb[t, pl.ds(jj * 16, 16)] = jnp.where(ok, lv, _TRASH)
                        pos = ch * _CHUNK + jj * 16 + lanes
                        sub = jnp.minimum(lax.div(pos, k), n_sub - 1)
                        subb[t, pl.ds(jj * 16, 16)] = sub
                return carry

            lax.fori_loop(0, n_tile_ch, decode_body, 0)

            def cb_body(cb, carry):
                newt_cb = newt_h.at[cb]

                def start_gather(t, buf):
                    ch = t * _NS + s

                    @pl.when(ch < nch)
                    def _():
                        pltpu.async_copy(newt_cb.at[subb.at[t]],
                                         rowsb.at[buf], sems[buf])

                def finish_chunk(t, buf):
                    ch = t * _NS + s

                    @pl.when(ch < nch)
                    def _():
                        pltpu.make_async_copy(newt_cb.at[subb.at[t]],
                                              rowsb.at[buf], sems[buf]).wait()
                        pltpu.sync_copy(rowsb.at[buf], acc.at[locb.at[t]],
                                        add=True)

                start_gather(0, 0)

                def chunk_body(p, carry2):
                    t = p * 2
                    start_gather(t + 1, 1)
                    finish_chunk(t, 0)
                    start_gather(t + 2, 0)
                    finish_chunk(t + 1, 1)
                    return carry2

                lax.fori_loop(0, _cdiv(n_tile_ch, 2), chunk_body, 0)
                plsc.subcore_barrier()
                pltpu.sync_copy(
                    acc.at[pl.ds(s * _ESTRIPE, _ESTRIPE), :],
                    s_h.at[pl.ds(base_e + s * _ESTRIPE, _ESTRIPE),
                           pl.ds(cb * 16, 16)])
                zero_stripe()
                plsc.subcore_barrier()
                return carry

            lax.fori_loop(0, 2 * _H // 16, cb_body, 0)

        for fi, (_, k, n_sub) in enumerate(_FAMS):
            run_family(idx_hbms[fi], newt_hbms[fi], s_hbms[fi], k, n_sub,
                       _spad(n_sub * k))

    return body(*idx_pads, *newts)


def _sc_repack(strips):
    mesh = plsc.VectorSubcoreMesh(core_axis_name="c", subcore_axis_name="s")
    out_type = [jax.ShapeDtypeStruct((_E, 2 * _H), jnp.float32) for _ in _FAMS]
    rp = 200
    scratch = [pltpu.VMEM((rp, 2 * _H), jnp.float32),
               pltpu.SemaphoreType.DMA]

    @functools.partial(pl.kernel, mesh=mesh, out_type=out_type,
                       scratch_types=scratch,
                       compiler_params=pltpu.CompilerParams(
                           use_tc_tiling_on_sc=False))
    def body(t0, t1, t2, t3, t4, t5, o0, o1, o2, o3, o4, o5, stag, sem):
        strip_hbms = (t0, t1, t2, t3, t4, t5)
        out_hbms = (o0, o1, o2, o3, o4, o5)
        c = lax.axis_index("c")
        s = lax.axis_index("s")
        w = c * _NS + s
        rows_per_w = _E // _NW

        def run_family(strip_h, out_h):
            def chunk_body(i, carry):
                r0 = w * rows_per_w + i * rp
                copies = [pltpu.async_copy(strip_h.at[cb, pl.ds(r0, rp), :],
                                           stag.at[:, pl.ds(cb * 16, 16)], sem)
                          for cb in range(16)]
                for cp in copies:
                    cp.wait()
                pltpu.sync_copy(stag, out_h.at[pl.ds(r0, rp), :])
                return carry

            lax.fori_loop(0, rows_per_w // rp, chunk_body, 0)

        for fi in range(len(_FAMS)):
            run_family(strip_hbms[fi], out_hbms[fi])

    return body(*strips)


def _tc_newf(g, rep, wab, k):
    n = g.shape[0]
    bn = 2000
    grid = n // bn

    def body(g_ref, rep_ref, wab_ref, new_ref, newt_ref):
        wmod = wab_ref[0:_H, :] + wab_ref[_H:2 * _H, :] * (1.0 / k)
        y = jnp.dot(g_ref[...], wmod, preferred_element_type=jnp.float32)
        new = jnp.concatenate([y, rep_ref[...]], axis=1)
        new_ref[...] = new
        for j in range(16):
            newt_ref[j, :, :] = new[:, j * 16:(j + 1) * 16]

    return pl.pallas_call(
        body,
        grid=(grid,),
        in_specs=[pl.BlockSpec((bn, _H), lambda i: (i, 0)),
                  pl.BlockSpec((bn, _H), lambda i: (i, 0)),
                  pl.BlockSpec((2 * _H, _H), lambda i: (0, 0))],
        out_specs=[pl.BlockSpec((bn, 2 * _H), lambda i: (i, 0)),
                   pl.BlockSpec((16, bn, 16), lambda i: (0, i, 0))],
        out_shape=[jax.ShapeDtypeStruct((n, 2 * _H), jnp.float32),
                   jax.ShapeDtypeStruct((16, n, 16), jnp.float32)],
    )(g, rep, wab)


def _tc_edge_mm1(edge_rep, s_fams, cnt, we1):
    be = 2000
    grid = _E // be

    def body(er_ref, s5_ref, s6_ref, p3_ref, p4_ref, p5_ref, p6_ref,
             cnt_ref, w_ref, y_ref, sum_ref, sq_ref):
        cnt_blk = cnt_ref[...]

        def recip(fi):
            cc = cnt_blk[:, fi:fi + 1]
            return 1.0 / jnp.maximum(cc, 1.0)

        s5 = s5_ref[...]
        s6 = s6_ref[...]
        p3 = p3_ref[...]
        p4 = p4_ref[...]
        p5 = p5_ref[...]
        p6 = p6_ref[...]
        cyc_s = s5 + s6
        cyc_n = s5 * recip(0) + s6 * recip(1)
        pat_s = p3 + p4 + p5 + p6
        pat_n = (p3 * recip(2) + p4 * recip(3)
                 + p5 * recip(4) + p6 * recip(5))
        w = w_ref[...]
        y = jnp.dot(er_ref[...], w[0:_H, :], preferred_element_type=jnp.float32)
        y += jnp.dot(cyc_s, w[_H:3 * _H, :], preferred_element_type=jnp.float32)
        y += jnp.dot(cyc_n, w[3 * _H:5 * _H, :], preferred_element_type=jnp.float32)
        y += jnp.dot(pat_s, w[5 * _H:7 * _H, :], preferred_element_type=jnp.float32)
        y += jnp.dot(pat_n, w[7 * _H:9 * _H, :], preferred_element_type=jnp.float32)
        y_ref[...] = y
        ps = jnp.sum(y, axis=0, keepdims=True)
        pq = jnp.sum(y * y, axis=0, keepdims=True)

        @pl.when(pl.program_id(0) == 0)
        def _():
            sum_ref[...] = jnp.zeros_like(sum_ref)
            sq_ref[...] = jnp.zeros_like(sq_ref)

        sum_ref[...] += jnp.broadcast_to(ps, sum_ref.shape)
        sq_ref[...] += jnp.broadcast_to(pq, sq_ref.shape)

    m = we1.shape[1]
    return pl.pallas_call(
        body,
        grid=(grid,),
        in_specs=[pl.BlockSpec((be, _H), lambda i: (i, 0))]
        + [pl.BlockSpec((be, 2 * _H), lambda i: (i, 0)) for _ in range(6)]
        + [pl.BlockSpec((be, 16), lambda i: (i, 0)),
           pl.BlockSpec((9 * _H, m), lambda i: (0, 0))],
        out_specs=[pl.BlockSpec((be, m), lambda i: (i, 0)),
                   pl.BlockSpec((8, m), lambda i: (0, 0)),
                   pl.BlockSpec((8, m), lambda i: (0, 0))],
        out_shape=[jax.ShapeDtypeStruct((_E, m), jnp.float32),
                   jax.ShapeDtypeStruct((8, m), jnp.float32),
                   jax.ShapeDtypeStruct((8, m), jnp.float32)],
    )(edge_rep, *s_fams, cnt, we1)


def _tc_mm_stats(x, w, bn):
    n, kdim = x.shape
    m = w.shape[1]
    grid = n // bn

    def body(x_ref, w_ref, y_ref, sum_ref, sq_ref):
        y = jnp.dot(x_ref[...], w_ref[...], preferred_element_type=jnp.float32)
        y_ref[...] = y
        ps = jnp.sum(y, axis=0, keepdims=True)
        pq = jnp.sum(y * y, axis=0, keepdims=True)

        @pl.when(pl.program_id(0) == 0)
        def _():
            sum_ref[...] = jnp.zeros_like(sum_ref)
            sq_ref[...] = jnp.zeros_like(sq_ref)

        sum_ref[...] += jnp.broadcast_to(ps, sum_ref.shape)
        sq_ref[...] += jnp.broadcast_to(pq, sq_ref.shape)

    return pl.pallas_call(
        body,
        grid=(grid,),
        in_specs=[pl.BlockSpec((bn, kdim), lambda i: (i, 0)),
                  pl.BlockSpec((kdim, m), lambda i: (0, 0))],
        out_specs=[pl.BlockSpec((bn, m), lambda i: (i, 0)),
                   pl.BlockSpec((8, m), lambda i: (0, 0)),
                   pl.BlockSpec((8, m), lambda i: (0, 0))],
        out_shape=[jax.ShapeDtypeStruct((n, m), jnp.float32),
                   jax.ShapeDtypeStruct((8, m), jnp.float32),
                   jax.ShapeDtypeStruct((8, m), jnp.float32)],
    )(x, w)


def _tc_bn_relu_mm_stats(y, ssum, ssq, g, b, w2, bn):
    n, kdim = y.shape
    m = w2.shape[1]
    grid = n // bn
    inv_n = 1.0 / n

    def body(y_ref, s_ref, q_ref, g_ref, b_ref, w_ref, o_ref, sum_ref, sq_ref):
        mu = s_ref[0:1, :] * inv_n
        var = q_ref[0:1, :] * inv_n - mu * mu
        h = (y_ref[...] - mu) * lax.rsqrt(var + 1e-5) * g_ref[0:1, :] + b_ref[0:1, :]
        h = jnp.maximum(h, 0.0)
        y2 = jnp.dot(h, w_ref[...], preferred_element_type=jnp.float32)
        o_ref[...] = y2
        ps = jnp.sum(y2, axis=0, keepdims=True)
        pq = jnp.sum(y2 * y2, axis=0, keepdims=True)

        @pl.when(pl.program_id(0) == 0)
        def _():
            sum_ref[...] = jnp.zeros_like(sum_ref)
            sq_ref[...] = jnp.zeros_like(sq_ref)

        sum_ref[...] += jnp.broadcast_to(ps, sum_ref.shape)
        sq_ref[...] += jnp.broadcast_to(pq, sq_ref.shape)

    return pl.pallas_call(
        body,
        grid=(grid,),
        in_specs=[pl.BlockSpec((bn, kdim), lambda i: (i, 0)),
                  pl.BlockSpec((8, kdim), lambda i: (0, 0)),
                  pl.BlockSpec((8, kdim), lambda i: (0, 0)),
                  pl.BlockSpec((1, kdim), lambda i: (0, 0)),
                  pl.BlockSpec((1, kdim), lambda i: (0, 0)),
                  pl.BlockSpec((kdim, m), lambda i: (0, 0))],
        out_specs=[pl.BlockSpec((bn, m), lambda i: (i, 0)),
                   pl.BlockSpec((8, m), lambda i: (0, 0)),
                   pl.BlockSpec((8, m), lambda i: (0, 0))],
        out_shape=[jax.ShapeDtypeStruct((n, m), jnp.float32),
                   jax.ShapeDtypeStruct((8, m), jnp.float32),
                   jax.ShapeDtypeStruct((8, m), jnp.float32)],
    )(y, ssum, ssq, g, b, w2)


def _tc_bn_relu(y, ssum, ssq, g, b, bn):
    n, m = y.shape
    grid = n // bn
    inv_n = 1.0 / n

    def body(y_ref, s_ref, q_ref, g_ref, b_ref, o_ref):
        mu = s_ref[0:1, :] * inv_n
        var = q_ref[0:1, :] * inv_n - mu * mu
        h = (y_ref[...] - mu) * lax.rsqrt(var + 1e-5) * g_ref[0:1, :] + b_ref[0:1, :]
        o_ref[...] = jnp.maximum(h, 0.0)

    return pl.pallas_call(
        body,
        grid=(grid,),
        in_specs=[pl.BlockSpec((bn, m), lambda i: (i, 0)),
                  pl.BlockSpec((8, m), lambda i: (0, 0)),
                  pl.BlockSpec((8, m), lambda i: (0, 0)),
                  pl.BlockSpec((1, m), lambda i: (0, 0)),
                  pl.BlockSpec((1, m), lambda i: (0, 0))],
        out_specs=pl.BlockSpec((bn, m), lambda i: (i, 0)),
        out_shape=jax.ShapeDtypeStruct((n, m), jnp.float32),
    )(y, ssum, ssq, g, b)


def _mlp(x, w1, g1, b1, w2, g2, b2, bn):
    y1, s1, q1 = _tc_mm_stats(x, w1, bn)
    y2, s2, q2 = _tc_bn_relu_mm_stats(y1, s1, q1, g1, b1, w2, bn)
    return _tc_bn_relu(y2, s2, q2, g2, b2, bn)


def kernel(edge_rep, cycle_rep_5, cycle_rep_6, path_rep_3, path_rep_4,
           path_rep_5, path_rep_6, Wab_c5, Wab_c6, Wab_p3, Wab_p4, Wab_p5,
           Wab_p6, We1, ge1, be1, We2, ge2, be2, Wc1, gc1, bc1, Wc2, gc2, bc2,
           Wp1, gp1, bp1, Wp2, gp2, bp2, cyc5_edge_idx, cyc6_edge_idx,
           path3_edge_idx, path4_edge_idx, path5_edge_idx, path6_edge_idx):
    reps = (cycle_rep_5, cycle_rep_6, path_rep_3, path_rep_4, path_rep_5,
            path_rep_6)
    wabs = (Wab_c5, Wab_c6, Wab_p3, Wab_p4, Wab_p5, Wab_p6)
    idxs = (cyc5_edge_idx, cyc6_edge_idx, path3_edge_idx, path4_edge_idx,
            path5_edge_idx, path6_edge_idx)

    idx_pads = []
    for idx, (_, k, n_sub) in zip(idxs, _FAMS):
        nslots = n_sub * k
        pad = _spad(nslots) - nslots
        idx_pads.append(jnp.concatenate(
            [idx, jnp.full((pad,), _E, jnp.int32)]) if pad else idx)

    gs = _sc_gather(edge_rep, idx_pads)
    cnt = _sc_counts(idx_pads)

    news, newts = [], []
    for g, rep, wab, (_, k, _n) in zip(gs, reps, wabs, _FAMS):
        new, newt = _tc_newf(g, rep, wab, k)
        news.append(new)
        newts.append(newt)

    s_fams = _sc_scatter(idx_pads, newts)

    y1, s1, q1 = _tc_edge_mm1(edge_rep, s_fams, cnt, We1)
    y2, s2, q2 = _tc_bn_relu_mm_stats(y1, s1, q1,
                                      ge1.reshape(1, -1), be1.reshape(1, -1),
                                      We2, 2000)
    edge_out = _tc_bn_relu(y2, s2, q2,
                           ge2.reshape(1, -1), be2.reshape(1, -1), 2000)

    cyc_outs = [_mlp(news[i], Wc1, gc1.reshape(1, -1), bc1.reshape(1, -1),
                     Wc2, gc2.reshape(1, -1), bc2.reshape(1, -1), 2000)
                for i in range(2)]
    pat_outs = [_mlp(news[i], Wp1, gp1.reshape(1, -1), bp1.reshape(1, -1),
                     Wp2, gp2.reshape(1, -1), bp2.reshape(1, -1), 2000)
                for i in range(2, 6)]

    return (edge_out,) + tuple(cyc_outs) + tuple(pat_outs)


# speedup vs baseline: 1.5799x; 1.2306x over previous
import functools

import jax
import jax.numpy as jnp
from jax import lax
from jax.experimental import pallas as pl
from jax.experimental.pallas import tpu as pltpu
from jax.experimental.pallas import tpu_sc as plsc

_H = 128
_E = 160000
_NS = 16
_NCORE = 2
_NW = _NS * _NCORE
_EOWN = _E // _NCORE
_ESTRIPE = _EOWN // _NS
_TRASH = _EOWN
_ACCROWS = _EOWN + 16
_CHUNK = 128
_GB = 40
_ZROWS = 500
_NRNG = 4
_RNG = _EOWN // _NRNG
_FLUSH = 1024
_FLB = 1280
_DRAIN = 1152
_LCAP = 8448

_FAMS = (("c5", 5, 20000), ("c6", 6, 20000),
         ("p3", 3, 10000), ("p4", 4, 10000),
         ("p5", 5, 10000), ("p6", 6, 10000))
_KS = (3, 4, 5, 6)


def _cdiv(a, b):
    return (a + b - 1) // b


def _spad(n_slots):
    return _cdiv(n_slots, _CHUNK) * _CHUNK


def _sc_gather(edge_rep, idx_pads):
    mesh = plsc.VectorSubcoreMesh(core_axis_name="c", subcore_axis_name="s")
    out_type = [jax.ShapeDtypeStruct((n, _H), jnp.float32) for _, _, n in _FAMS]
    scratch = ([pltpu.VMEM((_GB * k, _H), jnp.float32) for k in _KS]
               + [pltpu.VMEM((_GB * k,), jnp.int32) for k in _KS]
               + [pltpu.VMEM((_GB, _H), jnp.float32),
                  pltpu.SemaphoreType.DMA])

    @functools.partial(pl.kernel, mesh=mesh, out_type=out_type,
                       scratch_types=scratch,
                       compiler_params=pltpu.CompilerParams(
                           use_tc_tiling_on_sc=False,
                           needs_layout_passes=False))
    def body(edge_hbm, i0, i1, i2, i3, i4, i5,
             g0, g1, g2, g3, g4, g5,
             r3, r4, r5, r6, x3, x4, x5, x6, gsum, sem):
        idx_hbms = (i0, i1, i2, i3, i4, i5)
        g_hbms = (g0, g1, g2, g3, g4, g5)
        rows_k = {3: r3, 4: r4, 5: r5, 6: r6}
        ix_k = {3: x3, 4: x4, 5: x5, 6: x6}
        c = lax.axis_index("c")
        s = lax.axis_index("s")
        w = c * _NS + s

        def run_family_gather(idx_h, g_h, k, n_sub):
            rows = rows_k[k]
            ig = ix_k[k]
            nslots = _GB * k
            nch = n_sub // _GB

            def chunk_body(t, carry):
                ch = t * _NW + w

                @pl.when(ch < nch)
                def _():
                    pltpu.sync_copy(idx_h.at[pl.ds(ch * nslots, nslots)], ig)
                    pltpu.async_copy(edge_hbm.at[ig], rows, sem).wait()

                    def sub_body(r, carry2):
                        b = r * k
                        for cc in range(_H // 16):
                            v = rows[b, pl.ds(cc * 16, 16)]
                            for j in range(1, k):
                                v = v + rows[b + j, pl.ds(cc * 16, 16)]
                            gsum[r, pl.ds(cc * 16, 16)] = v
                        return carry2

                    lax.fori_loop(0, _GB, sub_body, 0)
                    pltpu.sync_copy(gsum, g_h.at[pl.ds(ch * _GB, _GB), :])

                return carry

            lax.fori_loop(0, _cdiv(nch, _NW), chunk_body, 0)

        for fi, (_, k, n_sub) in enumerate(_FAMS):
            run_family_gather(idx_hbms[fi], g_hbms[fi], k, n_sub)

    return body(edge_rep, *idx_pads)


def _sc_counts(idx_pads):
    mesh = plsc.VectorSubcoreMesh(core_axis_name="c", subcore_axis_name="s")
    out_type = jax.ShapeDtypeStruct((_E, 16), jnp.float32)
    scratch = [pltpu.VMEM((_CHUNK,), jnp.int32),
               pltpu.VMEM((_CHUNK,), jnp.int32),
               pltpu.VMEM((_CHUNK, 16), jnp.float32),
               pltpu.VMEM((_ZROWS, 16), jnp.float32),
               pltpu.VMEM_SHARED((_ACCROWS, 16), jnp.float32),
               pltpu.SemaphoreType.DMA]

    @functools.partial(pl.kernel, mesh=mesh, out_type=out_type,
                       scratch_types=scratch,
                       compiler_params=pltpu.CompilerParams(
                           use_tc_tiling_on_sc=False,
                           needs_layout_passes=False))
    def body(i0, i1, i2, i3, i4, i5, cnt_hbm,
             idxc, locc, onesb, zbuf, acc, sem):
        idx_hbms = (i0, i1, i2, i3, i4, i5)
        c = lax.axis_index("c")
        s = lax.axis_index("s")
        base_e = c * _EOWN
        lanes = lax.iota(jnp.int32, 16)
        zero_row = jnp.zeros((16,), jnp.float32)

        def fill_z(i, carry):
            zbuf[i, :] = zero_row
            return carry

        lax.fori_loop(0, _ZROWS, fill_z, 0)

        def zero_stripe():
            for t in range(_ESTRIPE // _ZROWS):
                pltpu.sync_copy(zbuf,
                                acc.at[pl.ds(s * _ESTRIPE + t * _ZROWS, _ZROWS), :])

        zero_stripe()
        plsc.subcore_barrier()

        def run_family_count(idx_h, fi, nslots_pad):
            one_hot = jnp.where(lanes == fi, 1.0, 0.0).astype(jnp.float32)

            def refill(i, carry):
                onesb[i, :] = one_hot
                return carry

            lax.fori_loop(0, _CHUNK, refill, 0)
            nch = nslots_pad // _CHUNK

            def chunk_body(t, carry):
                ch = t * _NS + s

                @pl.when(ch < nch)
                def _():
                    pltpu.sync_copy(idx_h.at[pl.ds(ch * _CHUNK, _CHUNK)], idxc)
                    for jj in range(_CHUNK // 16):
                        v = idxc[pl.ds(jj * 16, 16)]
                        lv = v - base_e
                        ok = (lv >= 0) & (lv < _EOWN)
                        locc[pl.ds(jj * 16, 16)] = jnp.where(ok, lv, _TRASH)
                    pltpu.sync_copy(onesb, acc.at[locc], add=True)

                return carry

            lax.fori_loop(0, _cdiv(nch, _NS), chunk_body, 0)

        for fi, (_, k, n_sub) in enumerate(_FAMS):
            run_family_count(idx_hbms[fi], fi, _spad(n_sub * k))
        plsc.subcore_barrier()
        pltpu.sync_copy(
            acc.at[pl.ds(s * _ESTRIPE, _ESTRIPE), :],
            cnt_hbm.at[pl.ds(base_e + s * _ESTRIPE, _ESTRIPE), :])

    return body(*idx_pads)


def _sc_scatter(idx_pads, newts):
    mesh = plsc.VectorSubcoreMesh(core_axis_name="c", subcore_axis_name="s")
    out_type = ([jax.ShapeDtypeStruct((_E, 2 * _H), jnp.float32) for _ in _FAMS]
                + [jax.ShapeDtypeStruct((_NCORE, _NS, _NRNG, _LCAP), jnp.int32),
                   jax.ShapeDtypeStruct((_NCORE, _NS, _NRNG, _LCAP), jnp.int32)])
    scratch = [pltpu.VMEM((_CHUNK,), jnp.int32)]
    scratch += [pltpu.VMEM((_FLB,), jnp.int32) for _ in range(2 * _NRNG)]
    scratch += [
               pltpu.VMEM((2, _CHUNK), jnp.int32),
               pltpu.VMEM((2, _CHUNK), jnp.int32),
               pltpu.VMEM((2, _CHUNK, 64), jnp.float32),
               pltpu.VMEM((208, 64), jnp.float32),
               pltpu.VMEM_SHARED((_RNG + 16, 64), jnp.float32),
               pltpu.SemaphoreType.DMA, pltpu.SemaphoreType.DMA,
               pltpu.SemaphoreType.DMA, pltpu.SemaphoreType.DMA]

    @functools.partial(pl.kernel, mesh=mesh, out_type=out_type,
                       scratch_types=scratch,
                       compiler_params=pltpu.CompilerParams(
                           use_tc_tiling_on_sc=False,
                           needs_layout_passes=False))
    def body(i0, i1, i2, i3, i4, i5, t0, t1, t2, t3, t4, t5,
             s0, s1, s2, s3, s4, s5, lloc_hbm, lsub_hbm,
             idxb, fl0, fl1, fl2, fl3, fs0, fs1, fs2, fs3, lb, sb, rb,
             zbuf, acc, seml0, seml1, semg0, semg1):
        flocs = (fl0, fl1, fl2, fl3)
        fsubs = (fs0, fs1, fs2, fs3)
        idx_hbms = (i0, i1, i2, i3, i4, i5)
        newt_hbms = (t0, t1, t2, t3, t4, t5)
        s_hbms = (s0, s1, s2, s3, s4, s5)
        semls = (seml0, seml1)
        semgs = (semg0, semg1)
        c = lax.axis_index("c")
        s = lax.axis_index("s")
        base_e = c * _EOWN
        lanes = lax.iota(jnp.int32, 16)
        zero16 = jnp.zeros((16,), jnp.float32)
        trash_v = jnp.full((16,), _RNG, jnp.int32)
        zsub_v = jnp.zeros((16,), jnp.int32)

        def fill_z(i, carry):
            for q in range(4):
                zbuf[i, pl.ds(q * 16, 16)] = zero16
            return carry

        lax.fori_loop(0, 208, fill_z, 0)

        def zero_acc():
            @pl.when(s < 15)
            def _():
                for t in range(6):
                    pltpu.sync_copy(zbuf.at[pl.ds(0, 208), :],
                                    acc.at[pl.ds(s * 1248 + t * 208, 208), :])
            pltpu.sync_copy(zbuf.at[pl.ds(0, 80), :],
                            acc.at[pl.ds(15 * 1248 + s * 80, 80), :])

        zero_acc()
        plsc.subcore_barrier()

        def run_family(idx_h, newt_h, s_h, k, n_sub, nslots_pad):
            nch = nslots_pad // _CHUNK
            n_my = (nch - 1 - s) // _NS + 1

            def chunk_body(t, carry):
                ch = t * _NS + s
                pltpu.sync_copy(idx_h.at[pl.ds(ch * _CHUNK, _CHUNK)], idxb)

                def group_body(jj, carry2):
                    cur = list(carry2[:_NRNG])
                    gof = list(carry2[_NRNG:])
                    v = idxb[pl.ds(jj * 16, 16)]
                    lv = v - base_e
                    pos = ch * _CHUNK + jj * 16 + lanes
                    sub = jnp.minimum(lax.div(pos, k), n_sub - 1)
                    for r in range(_NRNG):
                        lo = r * _RNG
                        ok = (lv >= lo) & (lv < lo + _RNG)
                        locr = lv - lo
                        n_r = jnp.sum(ok.astype(jnp.int32))
                        plsc.store_compressed(flocs[r].at[pl.ds(cur[r], 16)],
                                              locr, mask=ok)
                        plsc.store_compressed(fsubs[r].at[pl.ds(cur[r], 16)],
                                              sub, mask=ok)
                        cur[r] = cur[r] + n_r
                        full = cur[r] >= _FLUSH

                        @pl.when(full)
                        def _(r=r, gof_r=pl.multiple_of(gof[r], _FLUSH)):
                            pltpu.sync_copy(
                                flocs[r].at[pl.ds(0, _FLUSH)],
                                lloc_hbm.at[c, s, r, pl.ds(gof_r, _FLUSH)])
                            pltpu.sync_copy(
                                fsubs[r].at[pl.ds(0, _FLUSH)],
                                lsub_hbm.at[c, s, r, pl.ds(gof_r, _FLUSH)])
                            tl = flocs[r][pl.ds(_FLUSH, 16)]
                            flocs[r][pl.ds(0, 16)] = tl
                            ts_ = fsubs[r][pl.ds(_FLUSH, 16)]
                            fsubs[r][pl.ds(0, 16)] = ts_

                        gof[r] = gof[r] + jnp.where(full, _FLUSH, 0)
                        cur[r] = cur[r] - jnp.where(full, _FLUSH, 0)
                    return tuple(cur) + tuple(gof)

                return lax.fori_loop(0, _CHUNK // 16, group_body, carry)

            z = jnp.int32(0)
            st = lax.fori_loop(0, n_my, chunk_body, (z,) * (2 * _NRNG))
            totals = []
            for r in range(_NRNG):
                cur_r = st[r]
                gof_r = pl.multiple_of(st[_NRNG + r], _FLUSH)
                for g in range(8):
                    flocs[r][pl.ds(cur_r + g * 16, 16)] = trash_v
                    fsubs[r][pl.ds(cur_r + g * 16, 16)] = zsub_v
                pltpu.sync_copy(flocs[r].at[pl.ds(0, _DRAIN)],
                                lloc_hbm.at[c, s, r, pl.ds(gof_r, _DRAIN)])
                pltpu.sync_copy(fsubs[r].at[pl.ds(0, _DRAIN)],
                                lsub_hbm.at[c, s, r, pl.ds(gof_r, _DRAIN)])
                totals.append(gof_r + ((cur_r + 127) // 128) * 128)

            def range_body(r, carry):
                total = totals[0]
                for rr in range(1, _NRNG):
                    total = jnp.where(r == rr, totals[rr], total)
                nch2 = total // _CHUNK

                def cb_body(cb2, carry2):
                    newt_cb = newt_h.at[cb2]

                    def list_load(t, buf):
                        @pl.when(t < nch2)
                        def _():
                            pltpu.async_copy(
                                lloc_hbm.at[c, s, r, pl.ds(t * _CHUNK, _CHUNK)],
                                lb.at[buf], semls[buf])
                            pltpu.async_copy(
                                lsub_hbm.at[c, s, r, pl.ds(t * _CHUNK, _CHUNK)],
                                sb.at[buf], semls[buf])

                    def wait_lists(t, buf):
                        @pl.when(t < nch2)
                        def _():
                            pltpu.make_async_copy(
                                lloc_hbm.at[c, s, r, pl.ds(t * _CHUNK, _CHUNK)],
                                lb.at[buf], semls[buf]).wait()
                            pltpu.make_async_copy(
                                lsub_hbm.at[c, s, r, pl.ds(t * _CHUNK, _CHUNK)],
                                sb.at[buf], semls[buf]).wait()

                    def gather_start(t, buf):
                        @pl.when(t < nch2)
                        def _():
                            pltpu.async_copy(newt_cb.at[sb.at[buf]],
                                             rb.at[buf], semgs[buf])

                    def finish(t, buf):
                        @pl.when(t < nch2)
                        def _():
                            pltpu.make_async_copy(newt_cb.at[sb.at[buf]],
                                                  rb.at[buf], semgs[buf]).wait()
                            pltpu.sync_copy(rb.at[buf], acc.at[lb.at[buf]],
                                            add=True)

                    list_load(0, 0)
                    wait_lists(0, 0)
                    gather_start(0, 0)
                    list_load(1, 1)

                    def pipe_body(p, carry3):
                        t = p * 2
                        wait_lists(t + 1, 1)
                        gather_start(t + 1, 1)
                        finish(t, 0)
                        list_load(t + 2, 0)
                        wait_lists(t + 2, 0)
                        gather_start(t + 2, 0)
                        finish(t + 1, 1)
                        list_load(t + 3, 1)
                        return carry3

                    lax.fori_loop(0, (nch2 + 1) // 2, pipe_body, 0)
                    plsc.subcore_barrier()
                    out_col = cb2 * 64

                    @pl.when(s < 15)
                    def _():
                        pltpu.sync_copy(
                            acc.at[pl.ds(s * 1248, 1248), :],
                            s_h.at[pl.ds(base_e + r * _RNG + s * 1248, 1248),
                                   pl.ds(out_col, 64)])

                    pltpu.sync_copy(
                        acc.at[pl.ds(15 * 1248 + s * 80, 80), :],
                        s_h.at[pl.ds(base_e + r * _RNG + 15 * 1248 + s * 80, 80),
                               pl.ds(out_col, 64)])
                    zero_acc()
                    plsc.subcore_barrier()
                    return carry2

                lax.fori_loop(0, (2 * _H) // 64, cb_body, 0)
                return carry

            lax.fori_loop(0, _NRNG, range_body, 0)

        for fi, (_, k, n_sub) in enumerate(_FAMS):
            run_family(idx_hbms[fi], newt_hbms[fi], s_hbms[fi], k, n_sub,
                       _spad(n_sub * k))

    return body(*idx_pads, *newts)[:6]


def _tc_newf(g, rep, wab, k):
    n = g.shape[0]
    bn = 2000
    grid = n // bn

    def body(g_ref, rep_ref, wab_ref, new_ref, newt_ref):
        wmod = wab_ref[0:_H, :] + wab_ref[_H:2 * _H, :] * (1.0 / k)
        y = jnp.dot(g_ref[...], wmod, preferred_element_type=jnp.float32)
        new = jnp.concatenate([y, rep_ref[...]], axis=1)
        new_ref[...] = new
        for j in range(4):
            newt_ref[j, :, :] = new[:, j * 64:(j + 1) * 64]

    return pl.pallas_call(
        body,
        grid=(grid,),
        in_specs=[pl.BlockSpec((bn, _H), lambda i: (i, 0)),
                  pl.BlockSpec((bn, _H), lambda i: (i, 0)),
                  pl.BlockSpec((2 * _H, _H), lambda i: (0, 0))],
        out_specs=[pl.BlockSpec((bn, 2 * _H), lambda i: (i, 0)),
                   pl.BlockSpec((4, bn, 64), lambda i: (0, i, 0))],
        out_shape=[jax.ShapeDtypeStruct((n, 2 * _H), jnp.float32),
                   jax.ShapeDtypeStruct((4, n, 64), jnp.float32)],
    )(g, rep, wab)


def _tc_edge_mm1(edge_rep, s_fams, cnt, we1):
    be = 2000
    grid = _E // be

    def body(er_ref, s5_ref, s6_ref, p3_ref, p4_ref, p5_ref, p6_ref,
             cnt_ref, w_ref, y_ref, sum_ref, sq_ref):
        cnt_blk = cnt_ref[...]

        def recip(fi):
            cc = cnt_blk[:, fi:fi + 1]
            return 1.0 / jnp.maximum(cc, 1.0)

        s5 = s5_ref[...]
        s6 = s6_ref[...]
        p3 = p3_ref[...]
        p4 = p4_ref[...]
        p5 = p5_ref[...]
        p6 = p6_ref[...]
        cyc_s = s5 + s6
        cyc_n = s5 * recip(0) + s6 * recip(1)
        pat_s = p3 + p4 + p5 + p6
        pat_n = (p3 * recip(2) + p4 * recip(3)
                 + p5 * recip(4) + p6 * recip(5))
        w = w_ref[...]
        y = jnp.dot(er_ref[...], w[0:_H, :], preferred_element_type=jnp.float32)
        y += jnp.dot(cyc_s, w[_H:3 * _H, :], preferred_element_type=jnp.float32)
        y += jnp.dot(cyc_n, w[3 * _H:5 * _H, :], preferred_element_type=jnp.float32)
        y += jnp.dot(pat_s, w[5 * _H:7 * _H, :], preferred_element_type=jnp.float32)
        y += jnp.dot(pat_n, w[7 * _H:9 * _H, :], preferred_element_type=jnp.float32)
        y_ref[...] = y
        ps = jnp.sum(y, axis=0, keepdims=True)
        pq = jnp.sum(y * y, axis=0, keepdims=True)

        @pl.when(pl.program_id(0) == 0)
        def _():
            sum_ref[...] = jnp.zeros_like(sum_ref)
            sq_ref[...] = jnp.zeros_like(sq_ref)

        sum_ref[...] += jnp.broadcast_to(ps, sum_ref.shape)
        sq_ref[...] += jnp.broadcast_to(pq, sq_ref.shape)

    m = we1.shape[1]
    return pl.pallas_call(
        body,
        grid=(grid,),
        in_specs=[pl.BlockSpec((be, _H), lambda i: (i, 0))]
        + [pl.BlockSpec((be, 2 * _H), lambda i: (i, 0)) for _ in range(6)]
        + [pl.BlockSpec((be, 16), lambda i: (i, 0)),
           pl.BlockSpec((9 * _H, m), lambda i: (0, 0))],
        out_specs=[pl.BlockSpec((be, m), lambda i: (i, 0)),
                   pl.BlockSpec((8, m), lambda i: (0, 0)),
                   pl.BlockSpec((8, m), lambda i: (0, 0))],
        out_shape=[jax.ShapeDtypeStruct((_E, m), jnp.float32),
                   jax.ShapeDtypeStruct((8, m), jnp.float32),
                   jax.ShapeDtypeStruct((8, m), jnp.float32)],
    )(edge_rep, *s_fams, cnt, we1)


def _tc_mm_stats(x, w, bn):
    n, kdim = x.shape
    m = w.shape[1]
    grid = n // bn

    def body(x_ref, w_ref, y_ref, sum_ref, sq_ref):
        y = jnp.dot(x_ref[...], w_ref[...], preferred_element_type=jnp.float32)
        y_ref[...] = y
        ps = jnp.sum(y, axis=0, keepdims=True)
        pq = jnp.sum(y * y, axis=0, keepdims=True)

        @pl.when(pl.program_id(0) == 0)
        def _():
            sum_ref[...] = jnp.zeros_like(sum_ref)
            sq_ref[...] = jnp.zeros_like(sq_ref)

        sum_ref[...] += jnp.broadcast_to(ps, sum_ref.shape)
        sq_ref[...] += jnp.broadcast_to(pq, sq_ref.shape)

    return pl.pallas_call(
        body,
        grid=(grid,),
        in_specs=[pl.BlockSpec((bn, kdim), lambda i: (i, 0)),
                  pl.BlockSpec((kdim, m), lambda i: (0, 0))],
        out_specs=[pl.BlockSpec((bn, m), lambda i: (i, 0)),
                   pl.BlockSpec((8, m), lambda i: (0, 0)),
                   pl.BlockSpec((8, m), lambda i: (0, 0))],
        out_shape=[jax.ShapeDtypeStruct((n, m), jnp.float32),
                   jax.ShapeDtypeStruct((8, m), jnp.float32),
                   jax.ShapeDtypeStruct((8, m), jnp.float32)],
    )(x, w)


def _tc_bn_relu_mm_stats(y, ssum, ssq, g, b, w2, bn):
    n, kdim = y.shape
    m = w2.shape[1]
    grid = n // bn
    inv_n = 1.0 / n

    def body(y_ref, s_ref, q_ref, g_ref, b_ref, w_ref, o_ref, sum_ref, sq_ref):
        mu = s_ref[0:1, :] * inv_n
        var = q_ref[0:1, :] * inv_n - mu * mu
        h = (y_ref[...] - mu) * lax.rsqrt(var + 1e-5) * g_ref[0:1, :] + b_ref[0:1, :]
        h = jnp.maximum(h, 0.0)
        y2 = jnp.dot(h, w_ref[...], preferred_element_type=jnp.float32)
        o_ref[...] = y2
        ps = jnp.sum(y2, axis=0, keepdims=True)
        pq = jnp.sum(y2 * y2, axis=0, keepdims=True)

        @pl.when(pl.program_id(0) == 0)
        def _():
            sum_ref[...] = jnp.zeros_like(sum_ref)
            sq_ref[...] = jnp.zeros_like(sq_ref)

        sum_ref[...] += jnp.broadcast_to(ps, sum_ref.shape)
        sq_ref[...] += jnp.broadcast_to(pq, sq_ref.shape)

    return pl.pallas_call(
        body,
        grid=(grid,),
        in_specs=[pl.BlockSpec((bn, kdim), lambda i: (i, 0)),
                  pl.BlockSpec((8, kdim), lambda i: (0, 0)),
                  pl.BlockSpec((8, kdim), lambda i: (0, 0)),
                  pl.BlockSpec((1, kdim), lambda i: (0, 0)),
                  pl.BlockSpec((1, kdim), lambda i: (0, 0)),
                  pl.BlockSpec((kdim, m), lambda i: (0, 0))],
        out_specs=[pl.BlockSpec((bn, m), lambda i: (i, 0)),
                   pl.BlockSpec((8, m), lambda i: (0, 0)),
                   pl.BlockSpec((8, m), lambda i: (0, 0))],
        out_shape=[jax.ShapeDtypeStruct((n, m), jnp.float32),
                   jax.ShapeDtypeStruct((8, m), jnp.float32),
                   jax.ShapeDtypeStruct((8, m), jnp.float32)],
    )(y, ssum, ssq, g, b, w2)


def _tc_bn_relu(y, ssum, ssq, g, b, bn):
    n, m = y.shape
    grid = n // bn
    inv_n = 1.0 / n

    def body(y_ref, s_ref, q_ref, g_ref, b_ref, o_ref):
        mu = s_ref[0:1, :] * inv_n
        var = q_ref[0:1, :] * inv_n - mu * mu
        h = (y_ref[...] - mu) * lax.rsqrt(var + 1e-5) * g_ref[0:1, :] + b_ref[0:1, :]
        o_ref[...] = jnp.maximum(h, 0.0)

    return pl.pallas_call(
        body,
        grid=(grid,),
        in_specs=[pl.BlockSpec((bn, m), lambda i: (i, 0)),
                  pl.BlockSpec((8, m), lambda i: (0, 0)),
                  pl.BlockSpec((8, m), lambda i: (0, 0)),
                  pl.BlockSpec((1, m), lambda i: (0, 0)),
                  pl.BlockSpec((1, m), lambda i: (0, 0))],
        out_specs=pl.BlockSpec((bn, m), lambda i: (i, 0)),
        out_shape=jax.ShapeDtypeStruct((n, m), jnp.float32),
    )(y, ssum, ssq, g, b)


def _mlp(x, w1, g1, b1, w2, g2, b2, bn):
    y1, s1, q1 = _tc_mm_stats(x, w1, bn)
    y2, s2, q2 = _tc_bn_relu_mm_stats(y1, s1, q1, g1, b1, w2, bn)
    return _tc_bn_relu(y2, s2, q2, g2, b2, bn)


def kernel(edge_rep, cycle_rep_5, cycle_rep_6, path_rep_3, path_rep_4,
           path_rep_5, path_rep_6, Wab_c5, Wab_c6, Wab_p3, Wab_p4, Wab_p5,
           Wab_p6, We1, ge1, be1, We2, ge2, be2, Wc1, gc1, bc1, Wc2, gc2, bc2,
           Wp1, gp1, bp1, Wp2, gp2, bp2, cyc5_edge_idx, cyc6_edge_idx,
           path3_edge_idx, path4_edge_idx, path5_edge_idx, path6_edge_idx):
    reps = (cycle_rep_5, cycle_rep_6, path_rep_3, path_rep_4, path_rep_5,
            path_rep_6)
    wabs = (Wab_c5, Wab_c6, Wab_p3, Wab_p4, Wab_p5, Wab_p6)
    idxs = (cyc5_edge_idx, cyc6_edge_idx, path3_edge_idx, path4_edge_idx,
            path5_edge_idx, path6_edge_idx)

    idx_pads = []
    for idx, (_, k, n_sub) in zip(idxs, _FAMS):
        nslots = n_sub * k
        pad = _spad(nslots) - nslots
        idx_pads.append(jnp.concatenate(
            [idx, jnp.full((pad,), _E, jnp.int32)]) if pad else idx)

    gs = _sc_gather(edge_rep, idx_pads)
    cnt = _sc_counts(idx_pads)

    news, newts = [], []
    for g, rep, wab, (_, k, _n) in zip(gs, reps, wabs, _FAMS):
        new, newt = _tc_newf(g, rep, wab, k)
        news.append(new)
        newts.append(newt)

    s_fams = _sc_scatter(idx_pads, newts)

    y1, s1, q1 = _tc_edge_mm1(edge_rep, s_fams, cnt, We1)
    y2, s2, q2 = _tc_bn_relu_mm_stats(y1, s1, q1,
                                      ge1.reshape(1, -1), be1.reshape(1, -1),
                                      We2, 2000)
    edge_out = _tc_bn_relu(y2, s2, q2,
                           ge2.reshape(1, -1), be2.reshape(1, -1), 2000)

    cyc_outs = [_mlp(news[i], Wc1, gc1.reshape(1, -1), bc1.reshape(1, -1),
                     Wc2, gc2.reshape(1, -1), bc2.reshape(1, -1), 2000)
                for i in range(2)]
    pat_outs = [_mlp(news[i], Wp1, gp1.reshape(1, -1), bp1.reshape(1, -1),
                     Wp2, gp2.reshape(1, -1), bp2.reshape(1, -1), 2000)
                for i in range(2, 6)]

    return (edge_out,) + tuple(cyc_outs) + tuple(pat_outs)
